# baseline (speedup 1.0000x reference)
import functools

import jax
import jax.numpy as jnp
from jax import lax
from jax.experimental import pallas as pl
from jax.experimental.pallas import tpu as pltpu
from jax.experimental.pallas import tpu_sc as plsc

NUM_FIELDS = 26
VOCAB = 100000
EMBED_DIM = 32
BATCH = 16384

_INFO = plsc.get_sparse_core_info()
_NC, _NS = _INFO.num_cores, _INFO.num_subcores
_NW = _NC * _NS
_BPW = BATCH // _NW
_CH = 256
_ROWS_PER_FIELD = VOCAB // 4


def _make_kernel():
    mesh = plsc.VectorSubcoreMesh(core_axis_name="c", subcore_axis_name="s")

    @functools.partial(
        pl.kernel,
        mesh=mesh,
        out_type=jax.ShapeDtypeStruct((NUM_FIELDS * EMBED_DIM, BATCH),
                                      jnp.float32),
        scratch_types=[
            pltpu.VMEM((NUM_FIELDS, _BPW), jnp.int32),
            pltpu.VMEM((NUM_FIELDS, _BPW), jnp.int32),
            pltpu.VMEM((_CH, 128), jnp.float32),
            pltpu.VMEM((_CH, 128), jnp.float32),
            pltpu.VMEM((EMBED_DIM, _CH), jnp.float32),
            pltpu.VMEM((EMBED_DIM, _CH), jnp.float32),
            pltpu.SemaphoreType.DMA,
            pltpu.SemaphoreType.DMA,
            pltpu.SemaphoreType.DMA,
            pltpu.SemaphoreType.DMA,
        ],
        compiler_params=pltpu.CompilerParams(use_tc_tiling_on_sc=True,
                                             needs_layout_passes=False),
    )
    def k(tab3_hbm, x_hbm, out_hbm, idx_v, cb_v, g0, g1, st0, st1,
          sg0, sg1, sw0, sw1):
        wid = lax.axis_index("s") * _NC + lax.axis_index("c")
        base = wid * _BPW
        pltpu.sync_copy(x_hbm.at[:, pl.ds(base, _BPW)], idx_v)

        for f in range(NUM_FIELDS):
            def pre(i, _, f=f):
                v = idx_v[f, pl.ds(i * 16, 16)]
                row = v >> 2
                idx_v[f, pl.ds(i * 16, 16)] = row
                cb_v[f, pl.ds(i * 16, 16)] = (v & 3) << 5
                return ()
            lax.fori_loop(0, _BPW // 16, pre, (), unroll=False)

        def fire(f, half, gbuf, sem):
            for j in range(2):
                pltpu.async_copy(
                    tab3_hbm.at[f].at[
                        idx_v.at[f, pl.ds(half * _CH + j * 128, 128)]],
                    gbuf.at[pl.ds(j * 128, 128)],
                    sem,
                )

        def drain_g(gbuf, sem):
            pltpu.make_async_copy(
                tab3_hbm.at[0].at[pl.ds(0, _CH)], gbuf, sem).wait()

        def drain_w(stbuf, sem):
            pltpu.make_async_copy(
                out_hbm.at[pl.ds(0, EMBED_DIM), pl.ds(0, _CH)], stbuf, sem
            ).wait()

        def extract(gbuf, stbuf, f, half):
            def lbody(l, _):
                for bblk in range(2):
                    lane0 = bblk * 128 + l * 16
                    cb16 = cb_v[f, pl.ds(half * _CH + lane0, 16)]
                    row16 = lax.iota(jnp.int32, 16) + lane0
                    for d in range(EMBED_DIM):
                        val = plsc.load_gather(gbuf, [row16, cb16 + d])
                        stbuf[d, pl.ds(lane0, 16)] = val
                return ()
            lax.fori_loop(0, 8, lbody, (), unroll=False)

        def write(stbuf, f, half, sem):
            pltpu.async_copy(
                stbuf,
                out_hbm.at[pl.ds(f * EMBED_DIM, EMBED_DIM),
                           pl.ds(base + half * _CH, _CH)],
                sem,
            )

        fire(0, 0, g0, sg0)

        def body(f, _):
            fire(f, 1, g1, sg1)
            drain_g(g0, sg0)

            @pl.when(f >= 1)
            def _():
                drain_w(st0, sw0)
            extract(g0, st0, f, 0)
            write(st0, f, 0, sw0)

            @pl.when(f + 1 < NUM_FIELDS)
            def _():
                fire(f + 1, 0, g0, sg0)
            drain_g(g1, sg1)

            @pl.when(f >= 1)
            def _():
                drain_w(st1, sw1)
            extract(g1, st1, f, 1)
            write(st1, f, 1, sw1)
            return ()

        lax.fori_loop(0, NUM_FIELDS, body, (), unroll=False)
        drain_w(st0, sw0)
        drain_w(st1, sw1)

    return k


_kern = _make_kernel()


def kernel(x, tables):
    tab3 = tables.reshape(NUM_FIELDS, VOCAB // 4, 4 * EMBED_DIM)
    x_t = x.astype(jnp.int32).T
    out_t = _kern(tab3, x_t)
    return out_t.T


# speedup vs baseline: 1.1434x; 1.1434x over previous
import functools

import jax
import jax.numpy as jnp
from jax import lax
from jax.experimental import pallas as pl
from jax.experimental.pallas import tpu as pltpu
from jax.experimental.pallas import tpu_sc as plsc

NUM_FIELDS = 26
VOCAB = 100000
EMBED_DIM = 32
BATCH = 16384

_INFO = plsc.get_sparse_core_info()
_NC, _NS = _INFO.num_cores, _INFO.num_subcores
_NW = _NC * _NS
_N = BATCH * NUM_FIELDS
_PER_W = _N // _NW
_IW = 128
_ROWS_PER_W = _PER_W // _IW
_GRP = 13
_STEPS = _ROWS_PER_W // _GRP
_CHUNK = _GRP * _IW


def _make_gather():
    mesh = plsc.VectorSubcoreMesh(core_axis_name="c", subcore_axis_name="s")

    @functools.partial(
        pl.kernel,
        mesh=mesh,
        out_type=jax.ShapeDtypeStruct((_N, EMBED_DIM), jnp.float32),
        scratch_types=[
            pltpu.VMEM((_ROWS_PER_W, _IW), jnp.int32),
            pltpu.VMEM((_CHUNK, EMBED_DIM), jnp.float32),
            pltpu.VMEM((_CHUNK, EMBED_DIM), jnp.float32),
            pltpu.SemaphoreType.DMA,
            pltpu.SemaphoreType.DMA,
            pltpu.SemaphoreType.DMA,
            pltpu.SemaphoreType.DMA,
        ],
        compiler_params=pltpu.CompilerParams(use_tc_tiling_on_sc=False),
    )
    def gather_kernel(tab_hbm, idx_hbm, out_hbm, idx_v, rows0, rows1,
                      sg0, sg1, sw0, sw1):
        wid = lax.axis_index("s") * _NC + lax.axis_index("c")
        pltpu.sync_copy(idx_hbm.at[pl.ds(wid * _ROWS_PER_W, _ROWS_PER_W)], idx_v)
        out_base = wid * _PER_W

        def fire(c, buf, sem):
            for j in range(_GRP):
                pltpu.async_copy(
                    tab_hbm.at[idx_v.at[c * _GRP + j]],
                    buf.at[pl.ds(j * _IW, _IW)],
                    sem,
                )

        def drain(buf, sem):
            pltpu.make_async_copy(out_hbm.at[pl.ds(0, _CHUNK)], buf, sem).wait()

        def write(c, buf, sem):
            pltpu.async_copy(
                buf, out_hbm.at[pl.ds(out_base + c * _CHUNK, _CHUNK)], sem
            )

        fire(0, rows0, sg0)

        def pair(k, _):
            c0 = 2 * k

            @pl.when(k >= 1)
            def _():
                drain(rows1, sw1)
            fire(c0 + 1, rows1, sg1)
            drain(rows0, sg0)
            write(c0, rows0, sw0)

            @pl.when(c0 + 2 < _STEPS)
            def _():
                drain(rows0, sw0)
                fire(c0 + 2, rows0, sg0)
            drain(rows1, sg1)
            write(c0 + 1, rows1, sw1)
            return ()

        lax.fori_loop(0, _STEPS // 2, pair, (), unroll=False)
        drain(rows0, sw0)
        drain(rows1, sw1)

    return gather_kernel


_gather = _make_gather()


def kernel(x, tables):
    tab_flat = tables.reshape(NUM_FIELDS * VOCAB, EMBED_DIM)
    offs = (jnp.arange(NUM_FIELDS, dtype=jnp.int32) * VOCAB)[None, :]
    idx = (x.astype(jnp.int32) + offs).reshape(_N // _IW, _IW)
    out = _gather(tab_flat, idx)
    return out.reshape(BATCH, NUM_FIELDS * EMBED_DIM)
